# Initial kernel scaffold; baseline (speedup 1.0000x reference)
import functools

import jax
import jax.numpy as jnp
from jax.experimental import pallas as pl
from jax.experimental.pallas import tpu as pltpu

_K = 32
_T = 10
_BISECT_ROUNDS = 22
_BR = 256


def _sinkhorn_kernel(nb, d_ref, v_out_ref, v_ref, z_ref, e_ref):
    t = pl.program_id(0)
    b = pl.program_id(1)

    @pl.when(jnp.logical_and(t == 0, b == 0))
    def _init():
        v_ref[...] = jnp.ones_like(v_ref)

    @pl.when(jnp.logical_and(t > 0, b == 0))
    def _finish_prev_iter():
        v_ref[...] = 1.0 / z_ref[...]

    @pl.when(b == 0)
    def _reset_colsum():
        z_ref[...] = jnp.zeros_like(z_ref)

    e = jnp.exp(-d_ref[...])
    e_ref[...] = e
    s = jnp.sum(e * v_ref[...], axis=1, keepdims=True)
    u = 1.0 / s
    z_ref[...] += jnp.sum(e_ref[...] * u, axis=0, keepdims=True)

    @pl.when(jnp.logical_and(t == _T - 1, b == nb - 1))
    def _finalize():
        v_out_ref[...] = 1.0 / z_ref[...]


def _select_kernel(v_ref, d_ref, adj_ref):
    lv = jnp.log(v_ref[...])
    s = lv - d_ref[...]
    hi = jnp.max(s, axis=1, keepdims=True)
    rng = jnp.max(lv) - jnp.min(lv) + 1.0
    lo = hi - rng
    for _ in range(_BISECT_ROUNDS):
        mid = 0.5 * (lo + hi)
        cnt = jnp.sum((s > mid).astype(jnp.float32), axis=1, keepdims=True)
        ge = cnt >= _K
        lo = jnp.where(ge, mid, lo)
        hi = jnp.where(ge, hi, mid)
    adj_ref[...] = (s > lo).astype(jnp.float32)


def kernel(distances):
    n = distances.shape[0]
    br = _BR
    nb = n // br

    v = pl.pallas_call(
        functools.partial(_sinkhorn_kernel, nb),
        grid=(_T, nb),
        in_specs=[pl.BlockSpec((br, n), lambda t, b: (b, 0))],
        out_specs=pl.BlockSpec((1, n), lambda t, b: (0, 0)),
        out_shape=jax.ShapeDtypeStruct((1, n), jnp.float32),
        scratch_shapes=[
            pltpu.VMEM((1, n), jnp.float32),
            pltpu.VMEM((1, n), jnp.float32),
            pltpu.VMEM((br, n), jnp.float32),
        ],
        compiler_params=pltpu.CompilerParams(
            dimension_semantics=("arbitrary", "arbitrary")),
    )(distances)

    adj = pl.pallas_call(
        _select_kernel,
        grid=(nb,),
        in_specs=[
            pl.BlockSpec((1, n), lambda b: (0, 0)),
            pl.BlockSpec((br, n), lambda b: (b, 0)),
        ],
        out_specs=pl.BlockSpec((br, n), lambda b: (b, 0)),
        out_shape=jax.ShapeDtypeStruct((n, n), jnp.float32),
        compiler_params=pltpu.CompilerParams(
            dimension_semantics=("arbitrary",)),
    )(v, distances)
    return adj


# speedup vs baseline: 6.1808x; 6.1808x over previous
import functools

import jax
import jax.numpy as jnp
from jax.experimental import pallas as pl
from jax.experimental.pallas import tpu as pltpu

_K = 32
_T = 10
_BISECT_ROUNDS = 28
_BR = 256


def _sinkhorn_kernel(nb, d_ref, v_out_ref, v_ref, z_ref, e_ref):
    t = pl.program_id(0)
    b = pl.program_id(1)

    @pl.when(jnp.logical_and(t == 0, b == 0))
    def _init():
        v_ref[...] = jnp.ones_like(v_ref)

    @pl.when(jnp.logical_and(t > 0, b == 0))
    def _finish_prev_iter():
        v_ref[...] = 1.0 / z_ref[...]

    @pl.when(b == 0)
    def _reset_colsum():
        z_ref[...] = jnp.zeros_like(z_ref)

    e = jnp.exp(-d_ref[...])
    e_ref[...] = e
    s = jnp.sum(e * v_ref[...], axis=1, keepdims=True)
    u = 1.0 / s
    z_ref[...] += jnp.sum(e_ref[...] * u, axis=0, keepdims=True)

    @pl.when(jnp.logical_and(t == _T - 1, b == nb - 1))
    def _finalize():
        v_out_ref[...] = 1.0 / z_ref[...]


def _select_kernel(v_ref, d_ref, adj_ref):
    s = jnp.exp(-d_ref[...]) * v_ref[...]
    hi = jnp.max(s, axis=1, keepdims=True)
    lo = jnp.zeros_like(hi)
    for _ in range(_BISECT_ROUNDS):
        mid = 0.5 * (lo + hi)
        cnt = jnp.sum((s > mid).astype(jnp.float32), axis=1, keepdims=True)
        ge = cnt >= _K
        lo = jnp.where(ge, mid, lo)
        hi = jnp.where(ge, hi, mid)
    adj_ref[...] = (s > lo).astype(jnp.float32)


def kernel(distances):
    n = distances.shape[0]
    br = _BR
    nb = n // br

    v = pl.pallas_call(
        functools.partial(_sinkhorn_kernel, nb),
        grid=(_T, nb),
        in_specs=[pl.BlockSpec((br, n), lambda t, b: (b, 0))],
        out_specs=pl.BlockSpec((1, n), lambda t, b: (0, 0)),
        out_shape=jax.ShapeDtypeStruct((1, n), jnp.float32),
        scratch_shapes=[
            pltpu.VMEM((1, n), jnp.float32),
            pltpu.VMEM((1, n), jnp.float32),
            pltpu.VMEM((br, n), jnp.float32),
        ],
        compiler_params=pltpu.CompilerParams(
            dimension_semantics=("arbitrary", "arbitrary")),
    )(distances)

    adj = pl.pallas_call(
        _select_kernel,
        grid=(nb,),
        in_specs=[
            pl.BlockSpec((1, n), lambda b: (0, 0)),
            pl.BlockSpec((br, n), lambda b: (b, 0)),
        ],
        out_specs=pl.BlockSpec((br, n), lambda b: (b, 0)),
        out_shape=jax.ShapeDtypeStruct((n, n), jnp.float32),
        compiler_params=pltpu.CompilerParams(
            dimension_semantics=("arbitrary",)),
    )(v, distances)
    return adj


# speedup vs baseline: 6.8856x; 1.1140x over previous
import functools

import jax
import jax.numpy as jnp
from jax.experimental import pallas as pl
from jax.experimental.pallas import tpu as pltpu

_K = 32
_T_BF = 4
_T_F32 = 5
_M_ROUNDS = 20
_S_ROUNDS = 10
_X_ROUNDS = 3
_BR = 256
_NEG_LOG2E = -1.4426950408889634


def _sinkhorn_first_kernel(nb, d_ref, e16_ref, v_out_ref, z_ref, e_ref):
    b = pl.program_id(0)

    @pl.when(b == 0)
    def _reset_colsum():
        z_ref[...] = jnp.zeros_like(z_ref)

    e = jnp.exp2(d_ref[...] * _NEG_LOG2E)
    e_ref[...] = e
    e16_ref[...] = e.astype(jnp.bfloat16)
    s = jnp.sum(e, axis=1, keepdims=True)
    u = 1.0 / s
    z_ref[...] += jnp.sum(e_ref[...] * u, axis=0, keepdims=True)

    @pl.when(b == nb - 1)
    def _finalize():
        v_out_ref[...] = 1.0 / z_ref[...]


def _sinkhorn_bf16_kernel(nb, vin_ref, e16_ref, v_out_ref, v_ref, z_ref, e_ref):
    t = pl.program_id(0)
    b = pl.program_id(1)

    @pl.when(jnp.logical_and(t == 0, b == 0))
    def _init():
        v_ref[...] = vin_ref[...]

    @pl.when(jnp.logical_and(t > 0, b == 0))
    def _finish_prev_iter():
        v_ref[...] = 1.0 / z_ref[...]

    @pl.when(b == 0)
    def _reset_colsum():
        z_ref[...] = jnp.zeros_like(z_ref)

    e = e16_ref[...].astype(jnp.float32)
    e_ref[...] = e
    s = jnp.sum(e * v_ref[...], axis=1, keepdims=True)
    u = 1.0 / s
    z_ref[...] += jnp.sum(e_ref[...] * u, axis=0, keepdims=True)

    @pl.when(jnp.logical_and(t == _T_BF - 1, b == nb - 1))
    def _finalize():
        v_out_ref[...] = 1.0 / z_ref[...]


def _sinkhorn_kernel(nb, vin_ref, d_ref, v_out_ref, v_ref, z_ref, e_ref):
    t = pl.program_id(0)
    b = pl.program_id(1)

    @pl.when(jnp.logical_and(t == 0, b == 0))
    def _init():
        v_ref[...] = vin_ref[...]

    @pl.when(jnp.logical_and(t > 0, b == 0))
    def _finish_prev_iter():
        v_ref[...] = 1.0 / z_ref[...]

    @pl.when(b == 0)
    def _reset_colsum():
        z_ref[...] = jnp.zeros_like(z_ref)

    e = jnp.exp2(d_ref[...] * _NEG_LOG2E)
    e_ref[...] = e
    s = jnp.sum(e * v_ref[...], axis=1, keepdims=True)
    u = 1.0 / s
    z_ref[...] += jnp.sum(e_ref[...] * u, axis=0, keepdims=True)

    @pl.when(jnp.logical_and(t == _T_F32 - 1, b == nb - 1))
    def _finalize():
        v_out_ref[...] = 1.0 / z_ref[...]


def _select_kernel(v_ref, d_ref, adj_ref):
    br, n = adj_ref.shape
    d = d_ref[...]
    dmin = jnp.min(d, axis=1, keepdims=True)
    s = jnp.exp2((dmin - d) * -_NEG_LOG2E) * v_ref[...]
    m = jnp.max(s.reshape(br, n // 256, 256), axis=1)
    hi0 = jnp.max(m, axis=1, keepdims=True)
    lo = jnp.zeros_like(hi0)
    hi = hi0
    for _ in range(_M_ROUNDS):
        mid = 0.5 * (lo + hi)
        cnt = jnp.sum((m > mid).astype(jnp.float32), axis=1, keepdims=True)
        ge = cnt >= _K
        lo = jnp.where(ge, mid, lo)
        hi = jnp.where(ge, hi, mid)
    hi = hi0
    for _ in range(_S_ROUNDS):
        mid = 0.5 * (lo + hi)
        cnt = jnp.sum((s > mid).astype(jnp.float32), axis=1, keepdims=True)
        ge = cnt >= _K
        lo = jnp.where(ge, mid, lo)
        hi = jnp.where(ge, hi, mid)
    cnt = jnp.sum((s > lo).astype(jnp.float32), axis=1, keepdims=True)
    big = jnp.float32(3.4e38)
    for _ in range(_X_ROUNDS):
        xmin = jnp.min(jnp.where(s > lo, s, big), axis=1, keepdims=True)
        over = cnt > _K
        lo = jnp.where(over, xmin, lo)
        cnt = cnt - jnp.where(over, 1.0, 0.0)
    adj_ref[...] = (s > lo).astype(jnp.float32)


def kernel(distances):
    n = distances.shape[0]
    br = _BR
    nb = n // br
    f32 = jnp.float32

    e16, v1 = pl.pallas_call(
        functools.partial(_sinkhorn_first_kernel, nb),
        grid=(nb,),
        in_specs=[pl.BlockSpec((br, n), lambda b: (b, 0))],
        out_specs=[
            pl.BlockSpec((br, n), lambda b: (b, 0)),
            pl.BlockSpec((1, n), lambda b: (0, 0)),
        ],
        out_shape=[
            jax.ShapeDtypeStruct((n, n), jnp.bfloat16),
            jax.ShapeDtypeStruct((1, n), f32),
        ],
        scratch_shapes=[
            pltpu.VMEM((1, n), f32),
            pltpu.VMEM((br, n), f32),
        ],
        compiler_params=pltpu.CompilerParams(
            dimension_semantics=("arbitrary",)),
    )(distances)

    v5 = pl.pallas_call(
        functools.partial(_sinkhorn_bf16_kernel, nb),
        grid=(_T_BF, nb),
        in_specs=[
            pl.BlockSpec((1, n), lambda t, b: (0, 0)),
            pl.BlockSpec((br, n), lambda t, b: (b, 0)),
        ],
        out_specs=pl.BlockSpec((1, n), lambda t, b: (0, 0)),
        out_shape=jax.ShapeDtypeStruct((1, n), f32),
        scratch_shapes=[
            pltpu.VMEM((1, n), f32),
            pltpu.VMEM((1, n), f32),
            pltpu.VMEM((br, n), f32),
        ],
        compiler_params=pltpu.CompilerParams(
            dimension_semantics=("arbitrary", "arbitrary")),
    )(v1, e16)

    v = pl.pallas_call(
        functools.partial(_sinkhorn_kernel, nb),
        grid=(_T_F32, nb),
        in_specs=[
            pl.BlockSpec((1, n), lambda t, b: (0, 0)),
            pl.BlockSpec((br, n), lambda t, b: (b, 0)),
        ],
        out_specs=pl.BlockSpec((1, n), lambda t, b: (0, 0)),
        out_shape=jax.ShapeDtypeStruct((1, n), f32),
        scratch_shapes=[
            pltpu.VMEM((1, n), f32),
            pltpu.VMEM((1, n), f32),
            pltpu.VMEM((br, n), f32),
        ],
        compiler_params=pltpu.CompilerParams(
            dimension_semantics=("arbitrary", "arbitrary")),
    )(v5, distances)

    adj = pl.pallas_call(
        _select_kernel,
        grid=(nb,),
        in_specs=[
            pl.BlockSpec((1, n), lambda b: (0, 0)),
            pl.BlockSpec((br, n), lambda b: (b, 0)),
        ],
        out_specs=pl.BlockSpec((br, n), lambda b: (b, 0)),
        out_shape=jax.ShapeDtypeStruct((n, n), f32),
        compiler_params=pltpu.CompilerParams(
            dimension_semantics=("arbitrary",)),
    )(v, distances)
    return adj
